# Initial kernel scaffold; baseline (speedup 1.0000x reference)
import jax
import jax.numpy as jnp
from jax.experimental import pallas as pl


def kernel(student, teacher, target):
    raise NotImplementedError("write your pallas kernel here")


# speedup vs baseline: 435.8819x; 435.8819x over previous
import functools

import jax
import jax.numpy as jnp
from jax import lax
from jax.experimental import pallas as pl
from jax.experimental.pallas import tpu as pltpu
from jax.experimental.pallas import tpu_sc as plsc

N, C = 512, 256
NUM_CORES = 2
NUM_SUBCORES = 16
NW = NUM_CORES * NUM_SUBCORES
RPW = N // NW
L = 16
NCHUNK = C // L

_LN2 = 0.6931471805599453
_SQRT2 = 1.4142135623730951


def _softlog(x):
    bits = plsc.bitcast(x, jnp.int32)
    e = ((bits >> 23) & 0xFF) - 127
    m = plsc.bitcast((bits & 0x007FFFFF) | 0x3F800000, jnp.float32)
    big = m > _SQRT2
    m = jnp.where(big, m * 0.5, m)
    e = jnp.where(big, e + 1, e)
    s = (m - 1.0) / (m + 1.0)
    s2 = s * s
    logm = 2.0 * s * (1.0 + s2 * (1.0 / 3.0 + s2 * (1.0 / 5.0 + s2 * (1.0 / 7.0))))
    return e.astype(jnp.float32) * _LN2 + logm


def _sc_partials(student, teacher, target):
    mesh = plsc.VectorSubcoreMesh(core_axis_name="c", subcore_axis_name="s")

    @functools.partial(
        pl.kernel,
        out_type=jax.ShapeDtypeStruct((NW, L), jnp.float32),
        mesh=mesh,
        compiler_params=pltpu.CompilerParams(needs_layout_passes=False),
        scratch_types=[
            pltpu.VMEM((RPW, C), jnp.float32),
            pltpu.VMEM((RPW, C), jnp.float32),
            pltpu.VMEM((RPW, C), jnp.float32),
            pltpu.VMEM((L,), jnp.float32),
        ],
    )
    def body(s_hbm, t_hbm, g_hbm, out_hbm, s_v, t_v, g_v, acc_v):
        wid = lax.axis_index("s") * NUM_CORES + lax.axis_index("c")
        base = wid * RPW
        pltpu.sync_copy(s_hbm.at[pl.ds(base, RPW)], s_v)
        pltpu.sync_copy(t_hbm.at[pl.ds(base, RPW)], t_v)
        pltpu.sync_copy(g_hbm.at[pl.ds(base, RPW)], g_v)

        neg_fill = jnp.float32(-jnp.inf)

        def row_body(i, acc):
            def max_body(j, carry):
                ms, mt = carry
                sl = pl.ds(j * L, L)
                sv = s_v[i, sl]
                tv = t_v[i, sl]
                neg = g_v[i, sl] == 0.0
                ms = jnp.maximum(ms, jnp.where(neg, sv, neg_fill))
                mt = jnp.maximum(mt, jnp.where(neg, tv, neg_fill))
                return ms, mt

            msv, mtv = lax.fori_loop(
                0, NCHUNK, max_body,
                (jnp.full((L,), neg_fill, jnp.float32),
                 jnp.full((L,), neg_fill, jnp.float32)),
            )
            m_s = jnp.max(msv)
            m_t = jnp.max(mtv)

            def sum_body(j, carry):
                Es, Et, Av = carry
                sl = pl.ds(j * L, L)
                sv = s_v[i, sl]
                tv = t_v[i, sl]
                neg = g_v[i, sl] == 0.0
                es = jnp.exp(sv - m_s)
                et = jnp.exp(tv - m_t)
                Es = Es + jnp.where(neg, es, 0.0)
                Et = Et + jnp.where(neg, et, 0.0)
                Av = Av + jnp.where(neg, et * (tv - sv), 0.0)
                return Es, Et, Av

            z = jnp.zeros((L,), jnp.float32)
            Esv, Etv, Avv = lax.fori_loop(0, NCHUNK, sum_body, (z, z, z))
            Es = jnp.sum(Esv)
            Et = jnp.sum(Etv)
            A = jnp.sum(Avv)

            def kl_body(j, acc):
                sl = pl.ds(j * L, L)
                sv = s_v[i, sl]
                tv = t_v[i, sl]
                pos = g_v[i, sl] == 1.0
                cs = jnp.maximum(m_s, sv)
                ct = jnp.maximum(m_t, tv)
                zs = Es * jnp.exp(m_s - cs) + jnp.exp(sv - cs)
                wt = jnp.exp(m_t - ct)
                xt = jnp.exp(tv - ct)
                zt = Et * wt + xt
                num = A * wt + xt * (tv - sv)
                kl = num / zt + _softlog(zs / zt) + (cs - ct)
                return acc + jnp.where(pos, kl, 0.0)

            return lax.fori_loop(0, NCHUNK, kl_body, acc)

        acc = lax.fori_loop(0, RPW, row_body, jnp.zeros((L,), jnp.float32))
        acc_v[...] = acc * jnp.float32(1.0 / N)
        pltpu.sync_copy(acc_v, out_hbm.at[wid])

    return body(student, teacher, target)


def kernel(student, teacher, target):
    partials = _sc_partials(student, teacher, target)
    return jnp.sum(partials)


# speedup vs baseline: 438.1278x; 1.0052x over previous
import functools

import jax
import jax.numpy as jnp
from jax import lax
from jax.experimental import pallas as pl
from jax.experimental.pallas import tpu as pltpu
from jax.experimental.pallas import tpu_sc as plsc

N, C = 512, 256
NUM_CORES = 2
NUM_SUBCORES = 16
NW = NUM_CORES * NUM_SUBCORES
RPW = N // NW
L = 16
NCHUNK = C // L

_LN2 = 0.6931471805599453
_LOG_COEF = (
    1.0,
    -0.500000536441803,
    0.3333345055580139,
    -0.24994175136089325,
    0.19982793927192688,
    -0.16819174587726593,
    0.14910875260829926,
    -0.11938634514808655,
    0.053567204624414444,
)


def _softlog(x):
    bits = plsc.bitcast(x, jnp.int32)
    eb = (bits + 0x00400000) >> 23
    scale = plsc.bitcast((254 - eb) << 23, jnp.float32)
    t = x * scale - 1.0
    p = jnp.full_like(t, _LOG_COEF[-1])
    for coef in _LOG_COEF[-2::-1]:
        p = p * t + coef
    return (eb - 127).astype(jnp.float32) * _LN2 + t * p


def _sc_partials(student, teacher, target):
    mesh = plsc.VectorSubcoreMesh(core_axis_name="c", subcore_axis_name="s")

    @functools.partial(
        pl.kernel,
        out_type=jax.ShapeDtypeStruct((NW, L), jnp.float32),
        mesh=mesh,
        compiler_params=pltpu.CompilerParams(needs_layout_passes=False),
        scratch_types=[
            pltpu.VMEM((RPW, C), jnp.float32),
            pltpu.VMEM((RPW, C), jnp.float32),
            pltpu.VMEM((RPW, C), jnp.float32),
            pltpu.VMEM((L,), jnp.float32),
        ],
    )
    def body(s_hbm, t_hbm, g_hbm, out_hbm, s_v, t_v, g_v, acc_v):
        wid = lax.axis_index("s") * NUM_CORES + lax.axis_index("c")
        base = wid * RPW
        pltpu.sync_copy(s_hbm.at[pl.ds(base, RPW)], s_v)
        pltpu.sync_copy(t_hbm.at[pl.ds(base, RPW)], t_v)
        pltpu.sync_copy(g_hbm.at[pl.ds(base, RPW)], g_v)

        neg_fill = jnp.float32(-jnp.inf)

        def row_body(i, acc):
            ms = [jnp.full((L,), neg_fill, jnp.float32) for _ in range(2)]
            mt = [jnp.full((L,), neg_fill, jnp.float32) for _ in range(2)]
            for j in range(NCHUNK):
                sl = pl.ds(j * L, L)
                neg = g_v[i, sl] == 0.0
                k = j & 1
                ms[k] = jnp.maximum(ms[k], jnp.where(neg, s_v[i, sl], neg_fill))
                mt[k] = jnp.maximum(mt[k], jnp.where(neg, t_v[i, sl], neg_fill))
            m_s = jnp.max(jnp.maximum(ms[0], ms[1]))
            m_t = jnp.max(jnp.maximum(mt[0], mt[1]))

            zero = jnp.zeros((L,), jnp.float32)
            Esl = [zero, zero]
            Etl = [zero, zero]
            Avl = [zero, zero]
            for j in range(NCHUNK):
                sl = pl.ds(j * L, L)
                sv = s_v[i, sl]
                tv = t_v[i, sl]
                neg = g_v[i, sl] == 0.0
                es = jnp.exp(sv - m_s)
                et = jnp.exp(tv - m_t)
                k = j & 1
                Esl[k] = Esl[k] + jnp.where(neg, es, 0.0)
                Etl[k] = Etl[k] + jnp.where(neg, et, 0.0)
                Avl[k] = Avl[k] + jnp.where(neg, et * (tv - sv), 0.0)
            Es = jnp.sum(Esl[0] + Esl[1])
            Et = jnp.sum(Etl[0] + Etl[1])
            A = jnp.sum(Avl[0] + Avl[1])

            kls = [zero, zero]
            for j in range(NCHUNK):
                sl = pl.ds(j * L, L)
                sv = s_v[i, sl]
                tv = t_v[i, sl]
                pos = g_v[i, sl] == 1.0
                cs = jnp.maximum(m_s, sv)
                ct = jnp.maximum(m_t, tv)
                zs = Es * jnp.exp(m_s - cs) + jnp.exp(sv - cs)
                wt = jnp.exp(m_t - ct)
                xt = jnp.exp(tv - ct)
                zt = Et * wt + xt
                num = A * wt + xt * (tv - sv)
                rzt = 1.0 / zt
                kl = num * rzt + _softlog(zs * rzt) + (cs - ct)
                k = j & 1
                kls[k] = kls[k] + jnp.where(pos, kl, 0.0)
            return acc + kls[0] + kls[1]

        acc = lax.fori_loop(0, RPW, row_body, jnp.zeros((L,), jnp.float32))
        acc_v[...] = acc * jnp.float32(1.0 / N)
        pltpu.sync_copy(acc_v, out_hbm.at[wid])

    return body(student, teacher, target)


def kernel(student, teacher, target):
    partials = _sc_partials(student, teacher, target)
    return jnp.sum(partials)
